# baseline (speedup 1.0000x reference)
import functools

import jax
import jax.numpy as jnp
from jax import lax
from jax.experimental import pallas as pl
from jax.experimental.pallas import tpu as pltpu
from jax.experimental.pallas import tpu_sc as plsc

B, T, H, VOCAB, SLOTS = 1024, 200, 64, 100000, 64
NCAND = T - 3
GRP = 128
BB = 64
VT = 6144


def _make_sc_gather(rows):
    info = plsc.get_sparse_core_info()
    nw = info.num_cores * info.num_subcores
    g_per_w = rows // (nw * GRP)
    mesh = plsc.VectorSubcoreMesh(core_axis_name="c", subcore_axis_name="s")

    @functools.partial(
        pl.kernel,
        mesh=mesh,
        out_type=jax.ShapeDtypeStruct((rows, H), jnp.float32),
        scratch_types=[
            pltpu.VMEM((g_per_w * GRP,), jnp.int32),
            pltpu.VMEM((GRP, H), jnp.float32),
            pltpu.VMEM((GRP, H), jnp.float32),
            pltpu.SemaphoreType.DMA,
            pltpu.SemaphoreType.DMA,
        ],
        compiler_params=pltpu.CompilerParams(use_tc_tiling_on_sc=False),
    )
    def gather_k(table_hbm, idx_hbm, out_hbm, idx_v, buf0, buf1, sem0, sem1):
        wid = lax.axis_index("s") * info.num_cores + lax.axis_index("c")
        rbase = wid * g_per_w * GRP
        pltpu.sync_copy(idx_hbm.at[pl.ds(rbase, g_per_w * GRP)], idx_v)

        bufs = (buf0, buf1)
        sems = (sem0, sem1)

        def start(g, buf, sem):
            pltpu.async_copy(table_hbm.at[idx_v.at[pl.ds(g * GRP, GRP)]],
                             buf, sem)

        def finish(g, buf, sem):
            pltpu.make_async_copy(table_hbm.at[idx_v.at[pl.ds(g * GRP, GRP)]],
                                  buf, sem).wait()
            pltpu.sync_copy(buf, out_hbm.at[pl.ds(rbase + g * GRP, GRP)])

        start(0, buf0, sem0)
        start(1, buf1, sem1)

        def body(i, carry):
            for b in range(2):
                g = 2 * i + b
                finish(g, bufs[b], sems[b])

                @pl.when(g + 2 < g_per_w)
                def _():
                    start(g + 2, bufs[b], sems[b])

            return carry

        lax.fori_loop(0, g_per_w // 2, body, 0)

    return gather_k


@functools.cache
def _sc_gather(rows):
    return _make_sc_gather(rows)


def _encoder_body(h_ref, w1_ref, b1_ref, w2_ref, b2_ref, gamma_ref, beta_ref,
                  wg_ref, bg_ref, wq_ref, bq_ref, ctx_ref):
    h2 = h_ref[...].reshape(BB * T, H)
    a = jnp.maximum(
        jnp.dot(h2, w1_ref[...], preferred_element_type=jnp.float32)
        + b1_ref[...][None, :], 0.0)
    ff = (jnp.dot(a, w2_ref[...], preferred_element_type=jnp.float32)
          + b2_ref[...][None, :])
    x = h2 + ff
    m = jnp.mean(x, axis=-1, keepdims=True)
    xc = x - m
    v = jnp.mean(xc * xc, axis=-1, keepdims=True)
    xn = xc / jnp.sqrt(v + 1e-5) * gamma_ref[...][None, :] + beta_ref[...][None, :]
    x3 = xn.reshape(BB, T, H)

    xb = x3.astype(jnp.bfloat16).astype(jnp.float32)
    wgb = wg_ref[...].astype(jnp.bfloat16).astype(jnp.float32)
    s = jnp.sum(xb * wgb[None, None, :], axis=-1) + bg_ref[...]
    t_idx = lax.broadcasted_iota(jnp.int32, (BB, T), 1)
    valid = t_idx < NCAND

    u = lax.bitcast_convert_type(s, jnp.uint32)
    k = u ^ jnp.where(u >> 31 > 0,
                      jnp.uint32(0xFFFFFFFF), jnp.uint32(0x80000000))
    k = jnp.where(valid, k, jnp.uint32(0))

    def bs_body(_, c):
        lo, hi = c
        mid = hi - (hi - lo) // jnp.uint32(2)
        cnt = jnp.sum((k >= mid).astype(jnp.int32), axis=1, keepdims=True)
        pred = cnt >= SLOTS
        return jnp.where(pred, mid, lo), jnp.where(pred, hi, mid - 1)

    lo0 = jnp.zeros((BB, 1), jnp.uint32)
    hi0 = jnp.full((BB, 1), 0xFFFFFFFF, jnp.uint32)
    tau, _ = lax.fori_loop(0, 32, bs_body, (lo0, hi0))
    sel = valid & (k >= tau)

    q = (jnp.dot(x3[:, T - 2, :], wq_ref[...],
                 preferred_element_type=jnp.float32) + bq_ref[...][None, :])
    qb = q.astype(jnp.bfloat16).astype(jnp.float32)
    r = jnp.sum(xb * qb[:, None, :], axis=-1)
    rm = jnp.max(jnp.where(sel, r, -1e30), axis=1, keepdims=True)
    e = jnp.where(sel, jnp.exp(r - rm), 0.0)
    attn = e / jnp.sum(e, axis=1, keepdims=True)
    ctx_ref[...] = jnp.sum(x3 * attn[:, :, None], axis=1)


def _encoder(h3, W1, b1, W2, b2, gamma, beta, wg, bg, Wq, bq):
    rep = lambda *shape: pl.BlockSpec(shape, lambda i: (0,) * len(shape))
    bchunk = h3.shape[0]
    return pl.pallas_call(
        _encoder_body,
        grid=(bchunk // BB,),
        in_specs=[
            pl.BlockSpec((BB, T, H), lambda i: (i, 0, 0)),
            rep(H, 2 * H), rep(2 * H), rep(2 * H, H), rep(H),
            rep(H), rep(H), rep(H), rep(1), rep(H, H), rep(H),
        ],
        out_specs=pl.BlockSpec((BB, H), lambda i: (i, 0)),
        out_shape=jax.ShapeDtypeStruct((bchunk, H), jnp.float32),
    )(h3, W1, b1, W2, b2, gamma, beta, wg, bg, Wq, bq)


def _proj_body(ctx_ref, wo_ref, bo_ref, out_ref):
    out_ref[...] = (
        jnp.dot(ctx_ref[...], wo_ref[...], preferred_element_type=jnp.float32)
        + bo_ref[...][None, :])


def _proj(ctx, Wo, bo):
    nvt = pl.cdiv(VOCAB, VT)
    return pl.pallas_call(
        _proj_body,
        grid=(nvt,),
        in_specs=[
            pl.BlockSpec((B, H), lambda j: (0, 0)),
            pl.BlockSpec((H, VT), lambda j: (0, j)),
            pl.BlockSpec((VT,), lambda j: (j,)),
        ],
        out_specs=pl.BlockSpec((B, VT), lambda j: (0, j)),
        out_shape=jax.ShapeDtypeStruct((B, VOCAB), jnp.float32),
    )(ctx, Wo, bo)


def kernel(seq, embed, W1, b1, W2, b2, gamma, beta, Wg, bg, Wq, bq, Wo, bo):
    idx = seq.reshape(-1).astype(jnp.int32)
    h_flat = _sc_gather(B * T)(embed, idx)
    h3 = h_flat.reshape(B, T, H)
    ctx = _encoder(h3, W1, b1, W2, b2, gamma, beta, Wg.reshape(H), bg, Wq, bq)
    return _proj(ctx, Wo, bo)


# speedup vs baseline: 1.0203x; 1.0203x over previous
import functools

import jax
import jax.numpy as jnp
from jax import lax
from jax.experimental import pallas as pl
from jax.experimental.pallas import tpu as pltpu
from jax.experimental.pallas import tpu_sc as plsc

B, T, H, VOCAB, SLOTS = 1024, 200, 64, 100000, 64
NCAND = T - 3
GRP = 128
BB = 128
VT = 6144


def _make_sc_gather(rows):
    info = plsc.get_sparse_core_info()
    nw = info.num_cores * info.num_subcores
    g_per_w = rows // (nw * GRP)
    mesh = plsc.VectorSubcoreMesh(core_axis_name="c", subcore_axis_name="s")

    @functools.partial(
        pl.kernel,
        mesh=mesh,
        out_type=jax.ShapeDtypeStruct((rows, H), jnp.float32),
        scratch_types=[
            pltpu.VMEM((g_per_w * GRP,), jnp.int32),
            pltpu.VMEM((GRP, H), jnp.float32),
            pltpu.VMEM((GRP, H), jnp.float32),
            pltpu.SemaphoreType.DMA,
            pltpu.SemaphoreType.DMA,
        ],
        compiler_params=pltpu.CompilerParams(use_tc_tiling_on_sc=False),
    )
    def gather_k(table_hbm, idx_hbm, out_hbm, idx_v, buf0, buf1, sem0, sem1):
        wid = lax.axis_index("s") * info.num_cores + lax.axis_index("c")
        rbase = wid * g_per_w * GRP
        pltpu.sync_copy(idx_hbm.at[pl.ds(rbase, g_per_w * GRP)], idx_v)

        bufs = (buf0, buf1)
        sems = (sem0, sem1)

        def start(g, buf, sem):
            pltpu.async_copy(table_hbm.at[idx_v.at[pl.ds(g * GRP, GRP)]],
                             buf, sem)

        def finish(g, buf, sem):
            pltpu.make_async_copy(table_hbm.at[idx_v.at[pl.ds(g * GRP, GRP)]],
                                  buf, sem).wait()
            pltpu.sync_copy(buf, out_hbm.at[pl.ds(rbase + g * GRP, GRP)])

        start(0, buf0, sem0)
        start(1, buf1, sem1)

        def body(i, carry):
            for b in range(2):
                g = 2 * i + b
                finish(g, bufs[b], sems[b])

                @pl.when(g + 2 < g_per_w)
                def _():
                    start(g + 2, bufs[b], sems[b])

            return carry

        lax.fori_loop(0, g_per_w // 2, body, 0)

    return gather_k


@functools.cache
def _sc_gather(rows):
    return _make_sc_gather(rows)


def _encoder_body(h_ref, w1_ref, b1_ref, w2_ref, b2_ref, gamma_ref, beta_ref,
                  wg_ref, bg_ref, wq_ref, bq_ref, ctx_ref):
    h2 = h_ref[...].reshape(BB * T, H)
    a = jnp.maximum(
        jnp.dot(h2, w1_ref[...], preferred_element_type=jnp.float32)
        + b1_ref[...][None, :], 0.0)
    ff = (jnp.dot(a, w2_ref[...], preferred_element_type=jnp.float32)
          + b2_ref[...][None, :])
    x = h2 + ff
    m = jnp.mean(x, axis=-1, keepdims=True)
    xc = x - m
    v = jnp.mean(xc * xc, axis=-1, keepdims=True)
    xn = xc / jnp.sqrt(v + 1e-5) * gamma_ref[...][None, :] + beta_ref[...][None, :]
    x3 = xn.reshape(BB, T, H)

    xb = x3.astype(jnp.bfloat16).astype(jnp.float32)
    wgb = wg_ref[...].astype(jnp.bfloat16).astype(jnp.float32)
    s = jnp.sum(xb * wgb[None, None, :], axis=-1) + bg_ref[...]
    t_idx = lax.broadcasted_iota(jnp.int32, (BB, T), 1)
    valid = t_idx < NCAND

    u = lax.bitcast_convert_type(s, jnp.uint32)
    k = u ^ jnp.where(u >> 31 > 0,
                      jnp.uint32(0xFFFFFFFF), jnp.uint32(0x80000000))
    k = jnp.where(valid, k, jnp.uint32(0))

    def bs_body(_, c):
        lo, hi = c
        mid = hi - (hi - lo) // jnp.uint32(2)
        cnt = jnp.sum((k >= mid).astype(jnp.int32), axis=1, keepdims=True)
        pred = cnt >= SLOTS
        return jnp.where(pred, mid, lo), jnp.where(pred, hi, mid - 1)

    lo0 = jnp.zeros((BB, 1), jnp.uint32)
    hi0 = jnp.full((BB, 1), 0xFFFFFFFF, jnp.uint32)
    tau, _ = lax.fori_loop(0, 32, bs_body, (lo0, hi0))
    sel = valid & (k >= tau)

    q = (jnp.dot(x3[:, T - 2, :], wq_ref[...],
                 preferred_element_type=jnp.float32) + bq_ref[...][None, :])
    qb = q.astype(jnp.bfloat16).astype(jnp.float32)
    r = jnp.sum(xb * qb[:, None, :], axis=-1)
    rm = jnp.max(jnp.where(sel, r, -1e30), axis=1, keepdims=True)
    e = jnp.where(sel, jnp.exp(r - rm), 0.0)
    attn = e / jnp.sum(e, axis=1, keepdims=True)
    ctx_ref[...] = jnp.sum(x3 * attn[:, :, None], axis=1)


def _encoder(h3, W1, b1, W2, b2, gamma, beta, wg, bg, Wq, bq):
    rep = lambda *shape: pl.BlockSpec(shape, lambda i: (0,) * len(shape))
    bchunk = h3.shape[0]
    return pl.pallas_call(
        _encoder_body,
        grid=(bchunk // BB,),
        in_specs=[
            pl.BlockSpec((BB, T, H), lambda i: (i, 0, 0)),
            rep(H, 2 * H), rep(2 * H), rep(2 * H, H), rep(H),
            rep(H), rep(H), rep(H), rep(1), rep(H, H), rep(H),
        ],
        out_specs=pl.BlockSpec((BB, H), lambda i: (i, 0)),
        out_shape=jax.ShapeDtypeStruct((bchunk, H), jnp.float32),
    )(h3, W1, b1, W2, b2, gamma, beta, wg, bg, Wq, bq)


def _proj_body(ctx_ref, wo_ref, bo_ref, out_ref):
    out_ref[...] = (
        jnp.dot(ctx_ref[...], wo_ref[...], preferred_element_type=jnp.float32)
        + bo_ref[...][None, :])


def _proj(ctx, Wo, bo):
    nvt = pl.cdiv(VOCAB, VT)
    return pl.pallas_call(
        _proj_body,
        grid=(nvt,),
        in_specs=[
            pl.BlockSpec((B, H), lambda j: (0, 0)),
            pl.BlockSpec((H, VT), lambda j: (0, j)),
            pl.BlockSpec((VT,), lambda j: (j,)),
        ],
        out_specs=pl.BlockSpec((B, VT), lambda j: (0, j)),
        out_shape=jax.ShapeDtypeStruct((B, VOCAB), jnp.float32),
    )(ctx, Wo, bo)


def kernel(seq, embed, W1, b1, W2, b2, gamma, beta, Wg, bg, Wq, bq, Wo, bo):
    idx = seq.reshape(-1).astype(jnp.int32)
    h_flat = _sc_gather(B * T)(embed, idx)
    h3 = h_flat.reshape(B, T, H)
    ctx = _encoder(h3, W1, b1, W2, b2, gamma, beta, Wg.reshape(H), bg, Wq, bq)
    return _proj(ctx, Wo, bo)
